# baseline (device time: 8091 ns/iter reference)
import jax
import jax.numpy as jnp
from jax import lax
from jax.experimental import pallas as pl
from jax.experimental.pallas import tpu as pltpu

M = 256


def kernel(x):
    m_per, n = x.shape

    def body(x_ref, out_ref, send_sems, recv_sems, copy_sem):
        my_x = lax.axis_index("x")
        my_y = lax.axis_index("y")
        my_z = lax.axis_index("z")
        other_y = 1 - my_y
        nbr = (my_x, other_y, my_z)

        barrier_sem = pltpu.get_barrier_semaphore()
        pl.semaphore_signal(
            barrier_sem, inc=1,
            device_id=nbr, device_id_type=pl.DeviceIdType.MESH,
        )
        pl.semaphore_wait(barrier_sem, 1)

        half = M // 2
        rdmas = []
        for k in range(2):
            rdma = pltpu.make_async_remote_copy(
                src_ref=x_ref.at[pl.ds(k * half, half), pl.ds(other_y * M, M)],
                dst_ref=out_ref.at[pl.ds(my_y * M + k * half, half), :],
                send_sem=send_sems.at[k],
                recv_sem=recv_sems.at[k],
                device_id=nbr,
                device_id_type=pl.DeviceIdType.MESH,
            )
            rdma.start()
            rdmas.append(rdma)

        local = pltpu.make_async_copy(
            x_ref.at[:, pl.ds(my_y * M, M)],
            out_ref.at[pl.ds(my_y * M, M), :],
            copy_sem,
        )
        local.start()
        local.wait()

        for rdma in rdmas:
            rdma.wait()

    return pl.pallas_call(
        body,
        out_shape=jax.ShapeDtypeStruct((2 * m_per, n // 2), x.dtype),
        in_specs=[pl.BlockSpec(memory_space=pltpu.VMEM)],
        out_specs=pl.BlockSpec(memory_space=pltpu.VMEM),
        scratch_shapes=[
            pltpu.SemaphoreType.DMA((2,)),
            pltpu.SemaphoreType.DMA((2,)),
            pltpu.SemaphoreType.DMA,
        ],
        compiler_params=pltpu.CompilerParams(collective_id=0),
    )(x)


# device time: 7954 ns/iter; 1.0172x vs baseline; 1.0172x over previous
import jax
import jax.numpy as jnp
from jax import lax
from jax.experimental import pallas as pl
from jax.experimental.pallas import tpu as pltpu

M = 256


def kernel(x):
    m_per, n = x.shape

    def body(x_ref, out_ref, stage_ref, stage_sem, local_sem, send_sem, recv_sem):
        my_x = lax.axis_index("x")
        my_y = lax.axis_index("y")
        my_z = lax.axis_index("z")
        other_y = 1 - my_y
        nbr = (my_x, other_y, my_z)

        stage_cp = pltpu.make_async_copy(
            x_ref.at[:, pl.ds(other_y * M, M)], stage_ref, stage_sem
        )
        stage_cp.start()

        local_cp = pltpu.make_async_copy(
            x_ref.at[:, pl.ds(my_y * M, M)],
            out_ref.at[pl.ds(my_y * M, M), :],
            local_sem,
        )
        local_cp.start()

        barrier_sem = pltpu.get_barrier_semaphore()
        pl.semaphore_signal(
            barrier_sem, inc=1,
            device_id=nbr, device_id_type=pl.DeviceIdType.MESH,
        )
        pl.semaphore_wait(barrier_sem, 1)
        stage_cp.wait()

        rdma = pltpu.make_async_remote_copy(
            src_ref=stage_ref,
            dst_ref=out_ref.at[pl.ds(my_y * M, M), :],
            send_sem=send_sem,
            recv_sem=recv_sem,
            device_id=nbr,
            device_id_type=pl.DeviceIdType.MESH,
        )
        rdma.start()

        local_cp.wait()
        rdma.wait()

    return pl.pallas_call(
        body,
        out_shape=jax.ShapeDtypeStruct((2 * m_per, n // 2), x.dtype),
        in_specs=[pl.BlockSpec(memory_space=pltpu.MemorySpace.HBM)],
        out_specs=pl.BlockSpec(memory_space=pltpu.MemorySpace.HBM),
        scratch_shapes=[
            pltpu.VMEM((M, M), x.dtype),
            pltpu.SemaphoreType.DMA,
            pltpu.SemaphoreType.DMA,
            pltpu.SemaphoreType.DMA,
            pltpu.SemaphoreType.DMA,
        ],
        compiler_params=pltpu.CompilerParams(collective_id=0),
    )(pltpu.with_memory_space_constraint(x, pltpu.MemorySpace.HBM))
